# baseline (speedup 1.0000x reference)
import jax
import jax.numpy as jnp
from jax.experimental import pallas as pl
from jax.experimental.pallas import tpu as pltpu

_N = 10000
_BM = 256


def _support1_kernel(x_ref, w1_ref, o_ref):
    s1 = jnp.dot(x_ref[...], w1_ref[...], preferred_element_type=jnp.float32)
    o_ref[...] = s1.astype(jnp.bfloat16)


def _layer1_kernel(adj_ref, s1_ref, b1_ref, w2_ref, o_ref):
    a = adj_ref[...].astype(jnp.bfloat16)
    h = jnp.dot(a, s1_ref[...], preferred_element_type=jnp.float32)
    h = jnp.maximum(h + b1_ref[...], 0.0)
    s2 = jnp.dot(h.astype(jnp.bfloat16), w2_ref[...],
                 preferred_element_type=jnp.float32)
    o_ref[...] = s2.astype(jnp.bfloat16)


def _layer2_kernel(adj_ref, s2_ref, b2_ref, o_ref):
    a = adj_ref[...].astype(jnp.bfloat16)
    o = jnp.dot(a, s2_ref[...], preferred_element_type=jnp.float32)
    o_ref[...] = jnp.maximum(o + b2_ref[...], 0.0)


def kernel(x, adj, W1, b1, W2, b2):
    n, f_in = x.shape
    h1 = W1.shape[1]
    h2 = W2.shape[1]
    b1_2d = b1.reshape(1, h1)
    b2_2d = b2.reshape(1, h2)
    w2_bf = W2.astype(jnp.bfloat16)

    s1 = pl.pallas_call(
        _support1_kernel,
        out_shape=jax.ShapeDtypeStruct((n, h1), jnp.bfloat16),
    )(x, W1)

    grid = (pl.cdiv(n, _BM),)
    row_spec = pl.BlockSpec((_BM, _N), lambda i: (i, 0))
    params = pltpu.CompilerParams(dimension_semantics=("parallel",))

    s2 = pl.pallas_call(
        _layer1_kernel,
        grid=grid,
        in_specs=[
            row_spec,
            pl.BlockSpec((_N, h1), lambda i: (0, 0)),
            pl.BlockSpec((1, h1), lambda i: (0, 0)),
            pl.BlockSpec((h1, h2), lambda i: (0, 0)),
        ],
        out_specs=pl.BlockSpec((_BM, h2), lambda i: (i, 0)),
        out_shape=jax.ShapeDtypeStruct((n, h2), jnp.bfloat16),
        compiler_params=params,
    )(adj, s1, b1_2d, w2_bf)

    out = pl.pallas_call(
        _layer2_kernel,
        grid=grid,
        in_specs=[
            row_spec,
            pl.BlockSpec((_N, h2), lambda i: (0, 0)),
            pl.BlockSpec((1, h2), lambda i: (0, 0)),
        ],
        out_specs=pl.BlockSpec((_BM, h2), lambda i: (i, 0)),
        out_shape=jax.ShapeDtypeStruct((n, h2), jnp.float32),
        compiler_params=params,
    )(adj, s2, b2_2d)

    return out


# speedup vs baseline: 1.0038x; 1.0038x over previous
import jax
import jax.numpy as jnp
from jax.experimental import pallas as pl
from jax.experimental.pallas import tpu as pltpu

_N = 10000
_BM = 400


def _fused_kernel(adj_ref, x_ref, w1_ref, b1_ref, w2_ref, b2_ref,
                  out_ref, s2_ref):
    p = pl.program_id(0)
    i = pl.program_id(1)
    a = adj_ref[...].astype(jnp.bfloat16)

    @pl.when(p == 0)
    def _phase0():
        t = jnp.dot(a, x_ref[...], preferred_element_type=jnp.float32)
        h = jnp.dot(t.astype(jnp.bfloat16), w1_ref[...],
                    preferred_element_type=jnp.float32)
        h = jnp.maximum(h + b1_ref[...], 0.0)
        s2 = jnp.dot(h.astype(jnp.bfloat16), w2_ref[...],
                     preferred_element_type=jnp.float32)
        s2_ref[pl.ds(i * _BM, _BM), :] = s2.astype(jnp.bfloat16)

    @pl.when(p == 1)
    def _phase1():
        o = jnp.dot(a, s2_ref[...], preferred_element_type=jnp.float32)
        out_ref[...] = jnp.maximum(o + b2_ref[...], 0.0)


def kernel(x, adj, W1, b1, W2, b2):
    n, f_in = x.shape
    h1 = W1.shape[1]
    h2 = W2.shape[1]

    x_bf = x.astype(jnp.bfloat16)
    w1_bf = W1.astype(jnp.bfloat16)
    w2_bf = W2.astype(jnp.bfloat16)
    b1_2d = b1.reshape(1, h1)
    b2_2d = b2.reshape(1, h2)

    grid = (2, n // _BM)
    out = pl.pallas_call(
        _fused_kernel,
        grid=grid,
        in_specs=[
            pl.BlockSpec((_BM, _N), lambda p, i: (i, 0)),
            pl.BlockSpec((_N, f_in), lambda p, i: (0, 0)),
            pl.BlockSpec((f_in, h1), lambda p, i: (0, 0)),
            pl.BlockSpec((1, h1), lambda p, i: (0, 0)),
            pl.BlockSpec((h1, h2), lambda p, i: (0, 0)),
            pl.BlockSpec((1, h2), lambda p, i: (0, 0)),
        ],
        out_specs=pl.BlockSpec((_BM, h2), lambda p, i: (i, 0)),
        out_shape=jax.ShapeDtypeStruct((n, h2), jnp.float32),
        scratch_shapes=[pltpu.VMEM((_N, h2), jnp.bfloat16)],
        compiler_params=pltpu.CompilerParams(
            dimension_semantics=("arbitrary", "arbitrary")),
    )(adj, x_bf, w1_bf, b1_2d, w2_bf, b2_2d)

    return out
